# baseline (speedup 1.0000x reference)
import functools

import jax
import jax.numpy as jnp
from jax import lax
from jax.experimental import pallas as pl
from jax.experimental.pallas import tpu as pltpu
from jax.experimental.pallas import tpu_sc as plsc

_NC, _NS = 2, 16
_NW = _NC * _NS
_NL = 16
_TABLE_BLK = 2000
_NBUF = 4


def _table_body(w_ref, g_ref, b_ref, wa_ref, y_ref):
    e = w_ref[...]
    mu = jnp.mean(e, axis=1, keepdims=True)
    xc = e - mu
    var = jnp.mean(xc * xc, axis=1, keepdims=True)
    en = xc * lax.rsqrt(var + 1e-5)
    g = en * g_ref[...]
    y = lax.dot_general(g, wa_ref[...], (((1,), (1,)), ((), ())),
                        precision=lax.Precision.HIGHEST,
                        preferred_element_type=jnp.float32)
    b2 = lax.dot_general(b_ref[...], wa_ref[...], (((1,), (1,)), ((), ())),
                         precision=lax.Precision.HIGHEST,
                         preferred_element_type=jnp.float32)
    y_ref[...] = y + b2


def _transform_table(W_emb, ln_gamma, ln_beta, W_align):
    V, D = W_emb.shape
    blk = _TABLE_BLK
    return pl.pallas_call(
        _table_body,
        grid=(V // blk,),
        in_specs=[
            pl.BlockSpec((blk, D), lambda i: (i, 0)),
            pl.BlockSpec((1, D), lambda i: (0, 0)),
            pl.BlockSpec((1, D), lambda i: (0, 0)),
            pl.BlockSpec((D, D), lambda i: (0, 0)),
        ],
        out_specs=pl.BlockSpec((blk, D), lambda i: (i, 0)),
        out_shape=jax.ShapeDtypeStruct((V, D), jnp.float32),
    )(W_emb, ln_gamma.reshape(1, D), ln_beta.reshape(1, D), W_align)


def _sc_assemble(Y, ids, num_ids_flat, xnum_b, mask_b, rminv_b, b2_b, S, n_num,
                 num_tok):
    B, T = ids.shape
    V, D = Y.shape
    NT = num_ids_flat.shape[0]
    NTH = NT // 2
    nb = B // _NW
    nj = D // _NL
    mesh = plsc.VectorSubcoreMesh(core_axis_name="c", subcore_axis_name="s",
                                  num_cores=_NC, num_subcores=_NS)

    @functools.partial(
        pl.kernel,
        out_type=jax.ShapeDtypeStruct((B, S, D), jnp.float32),
        mesh=mesh,
        scratch_types=[
            pltpu.VMEM((nb, T), jnp.int32),
            pltpu.VMEM((NT,), jnp.int32),
            pltpu.VMEM((n_num, D), jnp.float32),
            pltpu.VMEM((n_num * num_tok * _NL,), jnp.float32),
            pltpu.VMEM((n_num * _NL,), jnp.float32),
            pltpu.VMEM((D,), jnp.float32),
            pltpu.VMEM((_NBUF * n_num * _NL,), jnp.float32),
            [pltpu.VMEM((S, D), jnp.float32)] * _NBUF,
            [pltpu.SemaphoreType.DMA] * _NBUF,
            [pltpu.SemaphoreType.DMA] * _NBUF,
            pltpu.SemaphoreType.DMA,
        ],
    )
    def k(ids_hbm, y_hbm, nids_hbm, xnb_hbm, mask_hbm, rminv_hbm, b2_hbm,
          out_hbm, ids_v, nids_v, z_v, mask_v, rminv_v, b2_v, xn_v,
          slab_v, gsem, ssem, psem):
        wid = lax.axis_index("s") * _NC + lax.axis_index("c")
        base = wid * nb

        pltpu.sync_copy(ids_hbm.at[pl.ds(base, nb)], ids_v)
        pltpu.sync_copy(nids_hbm, nids_v)
        pltpu.sync_copy(mask_hbm, mask_v)
        pltpu.sync_copy(rminv_hbm, rminv_v)
        pltpu.sync_copy(b2_hbm, b2_v)

        pltpu.async_copy(y_hbm.at[nids_v.at[pl.ds(0, NTH)]],
                         slab_v[0].at[pl.ds(0, NTH)], psem).wait()
        pltpu.async_copy(y_hbm.at[nids_v.at[pl.ds(NTH, NTH)]],
                         slab_v[1].at[pl.ds(0, NTH)], psem).wait()
        ih = NTH // num_tok

        def z_row_in(g_ref, row_off):
            def z_row(i, carry):
                for j in range(nj):
                    acc = jnp.zeros((_NL,), jnp.float32)
                    for t in range(num_tok):
                        acc = acc + mask_v[pl.ds((i * num_tok + t) * _NL, _NL)] * g_ref[
                            i * num_tok + t - row_off, pl.ds(j * _NL, _NL)]
                    z_v[i, pl.ds(j * _NL, _NL)] = acc * rminv_v[pl.ds(i * _NL, _NL)]
                return carry
            return z_row

        lax.fori_loop(0, ih, z_row_in(slab_v[0], 0), 0)
        lax.fori_loop(ih, n_num, z_row_in(slab_v[1], NTH), 0)

        def fire_gather(buf, i):
            pltpu.async_copy(y_hbm.at[ids_v.at[i]],
                             slab_v[buf].at[pl.ds(n_num, T)], gsem[buf])

        def wait_gather(buf, i):
            pltpu.make_async_copy(y_hbm.at[ids_v.at[i]],
                                  slab_v[buf].at[pl.ds(n_num, T)],
                                  gsem[buf]).wait()

        def fire_store(buf, i):
            pltpu.async_copy(slab_v[buf], out_hbm.at[base + i], ssem[buf])

        def wait_store(buf, i):
            pltpu.make_async_copy(slab_v[buf], out_hbm.at[base + i],
                                  ssem[buf]).wait()

        def num_rows(buf, kb):
            def body(i, carry):
                xv = xn_v[pl.ds((kb * n_num + i) * _NL, _NL)]
                for j in range(nj):
                    slab_v[buf][i, pl.ds(j * _NL, _NL)] = (
                        xv * z_v[i, pl.ds(j * _NL, _NL)] + b2_v[pl.ds(j * _NL, _NL)])
                return carry

            lax.fori_loop(0, n_num, body, 0)

        cw = n_num * _NL
        rem = nb % _NBUF
        if rem:
            pltpu.sync_copy(xnb_hbm.at[pl.ds(base * cw, rem * cw)],
                            xn_v.at[pl.ds(0, rem * cw)])
            for r in range(rem):
                fire_gather(0, r)
                wait_gather(0, r)
                num_rows(0, r)
                fire_store(0, r)
                wait_store(0, r)

        for kb in range(_NBUF):
            fire_gather(kb, rem + kb)

        def chunk(c, carry):
            i0 = rem + c * _NBUF
            pltpu.sync_copy(xnb_hbm.at[pl.ds((base + i0) * cw, _NBUF * cw)],
                            xn_v)
            for kb in range(_NBUF):
                wait_gather(kb, i0 + kb)
                num_rows(kb, kb)
                fire_store(kb, i0 + kb)
            for kb in range(_NBUF):
                nxt = i0 + kb + _NBUF

                @pl.when(nxt < nb)
                def _():
                    wait_store(kb, i0 + kb)
                    fire_gather(kb, nxt)

            return carry

        lax.fori_loop(0, (nb - rem) // _NBUF, chunk, 0)
        for kb in range(_NBUF):
            wait_store(kb, nb - _NBUF + kb)

    return k(ids, Y, num_ids_flat, xnum_b, mask_b, rminv_b, b2_b)


def kernel(x_num, num_col_input_ids, num_att_mask, x_cat_input_ids, cat_att_mask,
           x_bin_input_ids, bin_att_mask, W_emb, ln_gamma, ln_beta, num_bias, W_align):
    B, n_num = x_num.shape
    V, D = W_emb.shape
    cat_len = x_cat_input_ids.shape[1]
    bin_len = x_bin_input_ids.shape[1]
    num_tok = num_col_input_ids.shape[1]
    S = n_num + cat_len + bin_len

    Y = _transform_table(W_emb, ln_gamma, ln_beta, W_align)

    ids = jnp.concatenate(
        [x_cat_input_ids.astype(jnp.int32), x_bin_input_ids.astype(jnp.int32)],
        axis=1)
    num_ids_flat = num_col_input_ids.reshape(-1).astype(jnp.int32)

    m = num_att_mask
    mask_b = jnp.broadcast_to(
        (m != 0).astype(jnp.float32)[:, :, None],
        (n_num, num_tok, _NL)).reshape(n_num * num_tok * _NL)
    rminv_b = jnp.broadcast_to((1.0 / m.sum(1))[:, None],
                               (n_num, _NL)).reshape(n_num * _NL)
    b2n = lax.dot_general(num_bias.reshape(1, D), W_align, (((1,), (1,)), ((), ())),
                          precision=lax.Precision.HIGHEST,
                          preferred_element_type=jnp.float32)
    b2_b = b2n.reshape(D)
    xnum_b = jnp.broadcast_to(x_num[:, :, None],
                              (B, n_num, _NL)).reshape(B * n_num * _NL)

    embedding = _sc_assemble(Y, ids, num_ids_flat, xnum_b, mask_b, rminv_b,
                             b2_b, S, n_num, num_tok)

    attention_mask = jnp.concatenate(
        [jnp.ones((B, n_num), jnp.float32), cat_att_mask, bin_att_mask], axis=1)
    return embedding, attention_mask


# speedup vs baseline: 1.1034x; 1.1034x over previous
import functools

import jax
import jax.numpy as jnp
from jax import lax
from jax.experimental import pallas as pl
from jax.experimental.pallas import tpu as pltpu
from jax.experimental.pallas import tpu_sc as plsc

_NC, _NS = 2, 16
_NW = _NC * _NS
_NL = 16
_TABLE_BLK = 2000
_NBUF = 4


def _table_body(w_ref, g_ref, b_ref, wa_ref, y_ref):
    e = w_ref[...]
    mu = jnp.mean(e, axis=1, keepdims=True)
    xc = e - mu
    var = jnp.mean(xc * xc, axis=1, keepdims=True)
    en = xc * lax.rsqrt(var + 1e-5)
    g = en * g_ref[...]
    y = lax.dot_general(g, wa_ref[...], (((1,), (1,)), ((), ())),
                        precision=lax.Precision.HIGHEST,
                        preferred_element_type=jnp.float32)
    b2 = lax.dot_general(b_ref[...], wa_ref[...], (((1,), (1,)), ((), ())),
                         precision=lax.Precision.HIGHEST,
                         preferred_element_type=jnp.float32)
    y_ref[...] = y + b2


def _transform_table(W_emb, ln_gamma, ln_beta, W_align):
    V, D = W_emb.shape
    blk = _TABLE_BLK
    return pl.pallas_call(
        _table_body,
        grid=(V // blk,),
        in_specs=[
            pl.BlockSpec((blk, D), lambda i: (i, 0)),
            pl.BlockSpec((1, D), lambda i: (0, 0)),
            pl.BlockSpec((1, D), lambda i: (0, 0)),
            pl.BlockSpec((D, D), lambda i: (0, 0)),
        ],
        out_specs=pl.BlockSpec((blk, D), lambda i: (i, 0)),
        out_shape=jax.ShapeDtypeStruct((V, D), jnp.float32),
    )(W_emb, ln_gamma.reshape(1, D), ln_beta.reshape(1, D), W_align)


def _sc_assemble(Y, ids, num_ids_flat, xnum, w, b2, S, n_num, num_tok):
    B, T = ids.shape
    V, D = Y.shape
    NT = num_ids_flat.shape[0]
    NTH = NT // 2
    nb = B // _NW
    nj = D // _NL
    mesh = plsc.VectorSubcoreMesh(core_axis_name="c", subcore_axis_name="s",
                                  num_cores=_NC, num_subcores=_NS)

    @functools.partial(
        pl.kernel,
        out_type=jax.ShapeDtypeStruct((B, S, D), jnp.float32),
        mesh=mesh,
        compiler_params=pltpu.CompilerParams(needs_layout_passes=False),
        scratch_types=[
            pltpu.VMEM((nb, T), jnp.int32),
            pltpu.VMEM((NT,), jnp.int32),
            pltpu.VMEM((n_num, D), jnp.float32),
            pltpu.VMEM((n_num, num_tok), jnp.float32),
            pltpu.VMEM((D,), jnp.float32),
            pltpu.VMEM((nb, n_num), jnp.float32),
            [pltpu.VMEM((S, D), jnp.float32)] * _NBUF,
            [pltpu.SemaphoreType.DMA] * _NBUF,
            [pltpu.SemaphoreType.DMA] * _NBUF,
            pltpu.SemaphoreType.DMA,
        ],
    )
    def k(ids_hbm, y_hbm, nids_hbm, xnum_hbm, w_hbm, b2_hbm,
          out_hbm, ids_v, nids_v, z_v, w_v, b2_v, xn_v,
          slab_v, gsem, ssem, psem):
        wid = lax.axis_index("s") * _NC + lax.axis_index("c")
        base = wid * nb

        def splat(ref, idxs):
            return plsc.load_gather(
                ref, [jnp.full((_NL,), ix, jnp.int32) for ix in idxs])

        pltpu.sync_copy(ids_hbm.at[pl.ds(base, nb)], ids_v)
        pltpu.sync_copy(xnum_hbm.at[pl.ds(base, nb)], xn_v)
        pltpu.sync_copy(nids_hbm, nids_v)
        pltpu.sync_copy(w_hbm, w_v)
        pltpu.sync_copy(b2_hbm, b2_v)

        pltpu.async_copy(y_hbm.at[nids_v.at[pl.ds(0, NTH)]],
                         slab_v[0].at[pl.ds(0, NTH)], psem).wait()
        pltpu.async_copy(y_hbm.at[nids_v.at[pl.ds(NTH, NTH)]],
                         slab_v[1].at[pl.ds(0, NTH)], psem).wait()
        ih = NTH // num_tok

        def z_row_in(g_ref, row_off):
            def z_row(i, carry):
                wv = [splat(w_v, (i, t)) for t in range(num_tok)]
                for j in range(nj):
                    acc = jnp.zeros((_NL,), jnp.float32)
                    for t in range(num_tok):
                        acc = acc + wv[t] * g_ref[
                            i * num_tok + t - row_off, pl.ds(j * _NL, _NL)]
                    z_v[i, pl.ds(j * _NL, _NL)] = acc
                return carry
            return z_row

        lax.fori_loop(0, ih, z_row_in(slab_v[0], 0), 0)
        lax.fori_loop(ih, n_num, z_row_in(slab_v[1], NTH), 0)

        def fire_gather(buf, i):
            pltpu.async_copy(y_hbm.at[ids_v.at[i]],
                             slab_v[buf].at[pl.ds(n_num, T)], gsem[buf])

        def wait_gather(buf, i):
            pltpu.make_async_copy(y_hbm.at[ids_v.at[i]],
                                  slab_v[buf].at[pl.ds(n_num, T)],
                                  gsem[buf]).wait()

        def fire_store(buf, i):
            pltpu.async_copy(slab_v[buf], out_hbm.at[base + i], ssem[buf])

        def wait_store(buf, i):
            pltpu.make_async_copy(slab_v[buf], out_hbm.at[base + i],
                                  ssem[buf]).wait()

        def num_rows(buf, ib):
            def body(i, carry):
                xv = splat(xn_v, (ib, i))
                for j in range(nj):
                    slab_v[buf][i, pl.ds(j * _NL, _NL)] = (
                        xv * z_v[i, pl.ds(j * _NL, _NL)] + b2_v[pl.ds(j * _NL, _NL)])
                return carry

            lax.fori_loop(0, n_num, body, 0)

        rem = nb % _NBUF
        if rem:
            for r in range(rem):
                fire_gather(0, r)
                wait_gather(0, r)
                num_rows(0, r)
                fire_store(0, r)
                wait_store(0, r)

        for kb in range(_NBUF):
            fire_gather(kb, rem + kb)

        def chunk(c, carry):
            i0 = rem + c * _NBUF
            for kb in range(_NBUF):
                wait_gather(kb, i0 + kb)
                num_rows(kb, i0 + kb)
                fire_store(kb, i0 + kb)
            for kb in range(_NBUF):
                nxt = i0 + kb + _NBUF

                @pl.when(nxt < nb)
                def _():
                    wait_store(kb, i0 + kb)
                    fire_gather(kb, nxt)

            return carry

        lax.fori_loop(0, (nb - rem) // _NBUF, chunk, 0)
        for kb in range(_NBUF):
            wait_store(kb, nb - _NBUF + kb)

    return k(ids, Y, num_ids_flat, xnum, w, b2)


def kernel(x_num, num_col_input_ids, num_att_mask, x_cat_input_ids, cat_att_mask,
           x_bin_input_ids, bin_att_mask, W_emb, ln_gamma, ln_beta, num_bias, W_align):
    B, n_num = x_num.shape
    V, D = W_emb.shape
    cat_len = x_cat_input_ids.shape[1]
    bin_len = x_bin_input_ids.shape[1]
    num_tok = num_col_input_ids.shape[1]
    S = n_num + cat_len + bin_len

    Y = _transform_table(W_emb, ln_gamma, ln_beta, W_align)

    ids = jnp.concatenate(
        [x_cat_input_ids.astype(jnp.int32), x_bin_input_ids.astype(jnp.int32)],
        axis=1)
    num_ids_flat = num_col_input_ids.reshape(-1).astype(jnp.int32)

    m = num_att_mask
    w = (m != 0).astype(jnp.float32) / m.sum(1, keepdims=True)
    b2n = lax.dot_general(num_bias.reshape(1, D), W_align, (((1,), (1,)), ((), ())),
                          precision=lax.Precision.HIGHEST,
                          preferred_element_type=jnp.float32)
    b2 = b2n.reshape(D)

    embedding = _sc_assemble(Y, ids, num_ids_flat, x_num, w, b2, S, n_num,
                             num_tok)

    attention_mask = jnp.concatenate(
        [jnp.ones((B, n_num), jnp.float32), cat_att_mask, bin_att_mask], axis=1)
    return embedding, attention_mask
